# Initial kernel scaffold; baseline (speedup 1.0000x reference)
import jax
import jax.numpy as jnp
from jax.experimental import pallas as pl


def kernel(x, W):
    raise NotImplementedError("write your pallas kernel here")


# speedup vs baseline: 2.0502x; 2.0502x over previous
import functools

import jax
import jax.numpy as jnp
import numpy as np
from jax import lax
from jax.experimental import pallas as pl
from jax.experimental.pallas import tpu as pltpu
from jax.experimental.pallas import tpu_sc as plsc

_VOCAB = 100000
_EMBED = 128
_MAXLEN = 512
_B = 1024
_L = 200

_NC = 2
_NS = 16
_NW = _NC * _NS
_SEQ_PER_W = _B // _NW
_LANES = 16
_VREGS_PER_ROW = _EMBED // _LANES


def _pe_table() -> jnp.ndarray:
    pe = np.zeros((_MAXLEN, _EMBED), dtype=np.float32)
    position = np.arange(0, _MAXLEN)[:, np.newaxis]
    div_term = np.exp(np.arange(0, _EMBED, 2) * -(np.log(10000.0) / _EMBED))
    pe[:, 0::2] = np.sin(position * div_term)
    pe[:, 1::2] = np.cos(position * div_term)
    return jnp.asarray(pe[:_L])


_MESH = plsc.VectorSubcoreMesh(core_axis_name="c", subcore_axis_name="s")


@functools.partial(
    pl.kernel,
    out_type=jax.ShapeDtypeStruct((_B * _L, _EMBED), jnp.float32),
    mesh=_MESH,
    scratch_types=[
        pltpu.VMEM((_L,), jnp.int32),
        pltpu.VMEM((_L, _EMBED), jnp.float32),
        pltpu.VMEM((_L, _EMBED), jnp.float32),
        pltpu.SemaphoreType.DMA,
    ],
)
def _embed_lookup(x_hbm, w_hbm, pe_hbm, out_hbm, idx_v, rows_v, pe_v, sem):
    wid = lax.axis_index("s") * _NC + lax.axis_index("c")
    pltpu.sync_copy(pe_hbm, pe_v)

    def seq_body(s, carry):
        base = (wid * _SEQ_PER_W + s) * _L
        pltpu.sync_copy(x_hbm.at[pl.ds(base, _L)], idx_v)
        pltpu.async_copy(w_hbm.at[idx_v], rows_v, sem).wait()

        def row_body(r, c2):
            for c in range(_VREGS_PER_ROW):
                sl = pl.ds(c * _LANES, _LANES)
                rows_v[r, sl] = rows_v[r, sl] + pe_v[r, sl]
            return c2

        lax.fori_loop(0, _L, row_body, 0, unroll=2)
        pltpu.sync_copy(rows_v, out_hbm.at[pl.ds(base, _L)])
        return carry

    lax.fori_loop(0, _SEQ_PER_W, seq_body, 0)


def kernel(x, W):
    out = _embed_lookup(x.reshape(-1), W, _pe_table())
    return out.reshape(_B, _L, _EMBED)


# speedup vs baseline: 2.5670x; 1.2521x over previous
import functools

import jax
import jax.numpy as jnp
import numpy as np
from jax import lax
from jax.experimental import pallas as pl
from jax.experimental.pallas import tpu as pltpu
from jax.experimental.pallas import tpu_sc as plsc

_VOCAB = 100000
_EMBED = 128
_MAXLEN = 512
_B = 1024
_L = 200

_NC = 2
_NS = 16
_NW = _NC * _NS
_ROWS = _B * _L
_RPW = _ROWS // _NW
_CHUNK = 128
_NCHUNK = _RPW // _CHUNK
_LANES = 16
_VPR = _EMBED // _LANES
_PE_PAD = _L + _CHUNK


def _pe_table() -> jnp.ndarray:
    pe = np.zeros((_MAXLEN, _EMBED), dtype=np.float32)
    position = np.arange(0, _MAXLEN)[:, np.newaxis]
    div_term = np.exp(np.arange(0, _EMBED, 2) * -(np.log(10000.0) / _EMBED))
    pe[:, 0::2] = np.sin(position * div_term)
    pe[:, 1::2] = np.cos(position * div_term)
    pe = pe[:_L]
    return jnp.asarray(np.concatenate([pe, pe[: _PE_PAD - _L]], axis=0))


_MESH = plsc.VectorSubcoreMesh(core_axis_name="c", subcore_axis_name="s")


@functools.partial(
    pl.kernel,
    out_type=jax.ShapeDtypeStruct((_ROWS, _EMBED), jnp.float32),
    mesh=_MESH,
    scratch_types=[
        pltpu.VMEM((2, _CHUNK), jnp.int32),
        pltpu.VMEM((2, _CHUNK, _EMBED), jnp.float32),
        pltpu.VMEM((2, _CHUNK, _EMBED), jnp.float32),
        pltpu.VMEM((_PE_PAD, _EMBED), jnp.float32),
        pltpu.SemaphoreType.DMA,
        pltpu.SemaphoreType.DMA,
        pltpu.SemaphoreType.DMA,
        pltpu.SemaphoreType.DMA,
        pltpu.SemaphoreType.DMA,
        pltpu.SemaphoreType.DMA,
    ],
)
def _embed_lookup(x_hbm, w_hbm, pe_hbm, out_hbm, idx_v, gbuf, obuf, pe_v,
                  isem0, isem1, gsem0, gsem1, wsem0, wsem1):
    wid = lax.axis_index("s") * _NC + lax.axis_index("c")
    base = wid * _RPW
    isems = (isem0, isem1)
    gsems = (gsem0, gsem1)
    wsems = (wsem0, wsem1)

    pltpu.sync_copy(pe_hbm, pe_v)
    for b in range(2):
        pltpu.sync_copy(x_hbm.at[pl.ds(base + b * _CHUNK, _CHUNK)],
                        idx_v.at[b])
        pltpu.async_copy(w_hbm.at[idx_v.at[b]], gbuf.at[b], gsems[b])

    def add_chunk(b, p0):
        gb, ob = gbuf.at[b], obuf.at[b]

        def row_body(r, c2):
            pr = p0 + r
            for c in range(_VPR):
                sl = pl.ds(c * _LANES, _LANES)
                ob[r, sl] = gb[r, sl] + pe_v[pr, sl]
            return c2

        lax.fori_loop(0, _CHUNK, row_body, 0, unroll=4)

    def round_body(t, carry):
        for b in range(2):
            k = 2 * t + b
            rbase = base + k * _CHUNK

            @pl.when(t >= 1)
            def _wait_prev_out():
                pltpu.make_async_copy(
                    obuf.at[b],
                    out_hbm.at[pl.ds(rbase - 2 * _CHUNK, _CHUNK)],
                    wsems[b]).wait()

            pltpu.make_async_copy(w_hbm.at[idx_v.at[b]], gbuf.at[b],
                                  gsems[b]).wait()

            @pl.when(t < (_NCHUNK // 2) - 1)
            def _prefetch_idx():
                pltpu.async_copy(
                    x_hbm.at[pl.ds(rbase + 2 * _CHUNK, _CHUNK)],
                    idx_v.at[b], isems[b])

            add_chunk(b, lax.rem(k * _CHUNK, _L))
            pltpu.async_copy(obuf.at[b], out_hbm.at[pl.ds(rbase, _CHUNK)],
                             wsems[b])

            @pl.when(t < (_NCHUNK // 2) - 1)
            def _next_gather():
                pltpu.make_async_copy(
                    x_hbm.at[pl.ds(rbase + 2 * _CHUNK, _CHUNK)],
                    idx_v.at[b], isems[b]).wait()
                pltpu.async_copy(w_hbm.at[idx_v.at[b]], gbuf.at[b], gsems[b])
        return carry

    lax.fori_loop(0, _NCHUNK // 2, round_body, 0)
    for b in range(2):
        last = base + (_NCHUNK - 2 + b) * _CHUNK
        pltpu.make_async_copy(obuf.at[b],
                              out_hbm.at[pl.ds(last, _CHUNK)],
                              wsems[b]).wait()


def kernel(x, W):
    out = _embed_lookup(x.reshape(-1), W, _pe_table())
    return out.reshape(_B, _L, _EMBED)


# speedup vs baseline: 2.7307x; 1.0638x over previous
import functools

import jax
import jax.numpy as jnp
import numpy as np
from jax import lax
from jax.experimental import pallas as pl
from jax.experimental.pallas import tpu as pltpu
from jax.experimental.pallas import tpu_sc as plsc

_VOCAB = 100000
_EMBED = 128
_MAXLEN = 512
_B = 1024
_L = 200

_NC = 2
_NS = 16
_NW = _NC * _NS
_ROWS = _B * _L
_RPW = _ROWS // _NW
_CHUNK = 64
_NCHUNK = _RPW // _CHUNK
_NBUF = 4
_NROUND = _NCHUNK // _NBUF
_PE_PAD = _L + _CHUNK


def _pe_table() -> jnp.ndarray:
    pe = np.zeros((_MAXLEN, _EMBED), dtype=np.float32)
    position = np.arange(0, _MAXLEN)[:, np.newaxis]
    div_term = np.exp(np.arange(0, _EMBED, 2) * -(np.log(10000.0) / _EMBED))
    pe[:, 0::2] = np.sin(position * div_term)
    pe[:, 1::2] = np.cos(position * div_term)
    pe = pe[:_L]
    return jnp.asarray(np.concatenate([pe, pe[: _PE_PAD - _L]], axis=0))


_MESH = plsc.VectorSubcoreMesh(core_axis_name="c", subcore_axis_name="s")


@functools.partial(
    pl.kernel,
    out_type=jax.ShapeDtypeStruct((_ROWS, _EMBED), jnp.float32),
    mesh=_MESH,
    scratch_types=[
        pltpu.VMEM((_NBUF, _CHUNK), jnp.int32),
        pltpu.VMEM((_NBUF, _CHUNK, _EMBED), jnp.float32),
        [pltpu.SemaphoreType.DMA] * _NBUF,
        [pltpu.SemaphoreType.DMA] * _NBUF,
        [pltpu.SemaphoreType.DMA] * _NBUF,
    ],
)
def _embed_lookup(x_hbm, w_hbm, pe_hbm, out_hbm, idx_v, buf,
                  isems, gsems, wsems):
    wid = lax.axis_index("s") * _NC + lax.axis_index("c")
    base = wid * _RPW

    def phase(k):
        return lax.rem(k * _CHUNK, _L)

    for b in range(_NBUF):
        pltpu.sync_copy(x_hbm.at[pl.ds(base + b * _CHUNK, _CHUNK)],
                        idx_v.at[b])
        pltpu.sync_copy(pe_hbm.at[pl.ds(phase(b), _CHUNK)], buf.at[b])
        pltpu.async_copy(w_hbm.at[idx_v.at[b]], buf.at[b], gsems[b],
                         add=True)

    def round_body(t, carry):
        for b in range(_NBUF):
            k = _NBUF * t + b
            rbase = base + k * _CHUNK
            pltpu.make_async_copy(w_hbm.at[idx_v.at[b]], buf.at[b],
                                  gsems[b]).wait()
            pltpu.async_copy(buf.at[b], out_hbm.at[pl.ds(rbase, _CHUNK)],
                             wsems[b])

            @pl.when(t < _NROUND - 1)
            def _prep_next():
                nbase = rbase + _NBUF * _CHUNK
                pltpu.async_copy(x_hbm.at[pl.ds(nbase, _CHUNK)],
                                 idx_v.at[b], isems[b])
                pltpu.make_async_copy(buf.at[b],
                                      out_hbm.at[pl.ds(rbase, _CHUNK)],
                                      wsems[b]).wait()
                pltpu.sync_copy(pe_hbm.at[pl.ds(phase(k + _NBUF), _CHUNK)],
                                buf.at[b])
                pltpu.make_async_copy(x_hbm.at[pl.ds(nbase, _CHUNK)],
                                      idx_v.at[b], isems[b]).wait()
                pltpu.async_copy(w_hbm.at[idx_v.at[b]], buf.at[b],
                                 gsems[b], add=True)
        return carry

    lax.fori_loop(0, _NROUND, round_body, 0)
    for b in range(_NBUF):
        last = base + (_NCHUNK - _NBUF + b) * _CHUNK
        pltpu.make_async_copy(buf.at[b], out_hbm.at[pl.ds(last, _CHUNK)],
                              wsems[b]).wait()


def kernel(x, W):
    out = _embed_lookup(x.reshape(-1), W, _pe_table())
    return out.reshape(_B, _L, _EMBED)


# speedup vs baseline: 6.7077x; 2.4564x over previous
import functools

import jax
import jax.numpy as jnp
import numpy as np
from jax import lax
from jax.experimental import pallas as pl
from jax.experimental.pallas import tpu as pltpu
from jax.experimental.pallas import tpu_sc as plsc

_VOCAB = 100000
_EMBED = 128
_MAXLEN = 512
_B = 1024
_L = 200

_NC = 2
_NS = 16
_NW = _NC * _NS
_ROWS = _B * _L
_RPW = _ROWS // _NW
_CHUNK = 64
_NCHUNK = _RPW // _CHUNK
_NBUF = 4
_NROUND = _NCHUNK // _NBUF
_PE_PAD = _L + _CHUNK


def _pe_table() -> jnp.ndarray:
    pe = np.zeros((_MAXLEN, _EMBED), dtype=np.float32)
    position = np.arange(0, _MAXLEN)[:, np.newaxis]
    div_term = np.exp(np.arange(0, _EMBED, 2) * -(np.log(10000.0) / _EMBED))
    pe[:, 0::2] = np.sin(position * div_term)
    pe[:, 1::2] = np.cos(position * div_term)
    pe = pe[:_L]
    return jnp.asarray(np.concatenate([pe, pe[: _PE_PAD - _L]], axis=0))


_MESH = plsc.VectorSubcoreMesh(core_axis_name="c", subcore_axis_name="s")


@functools.partial(
    pl.kernel,
    out_type=jax.ShapeDtypeStruct((_ROWS, _EMBED), jnp.float32),
    mesh=_MESH,
    scratch_types=[
        pltpu.VMEM((_NBUF, _CHUNK), jnp.int32),
        pltpu.VMEM((_NBUF, _CHUNK, _EMBED), jnp.float32),
        pltpu.VMEM_SHARED((_PE_PAD, _EMBED), jnp.float32),
        [pltpu.SemaphoreType.DMA] * _NBUF,
        [pltpu.SemaphoreType.DMA] * _NBUF,
        [pltpu.SemaphoreType.DMA] * _NBUF,
    ],
)
def _embed_lookup(x_hbm, w_hbm, pe_hbm, out_hbm, idx_v, buf, pe_sh,
                  isems, gsems, wsems):
    wid = lax.axis_index("s") * _NC + lax.axis_index("c")
    base = wid * _RPW

    def phase(k):
        return lax.rem(k * _CHUNK, _L)

    @pl.when(lax.axis_index("s") == 0)
    def _stage_pe():
        pltpu.sync_copy(pe_hbm, pe_sh)

    plsc.subcore_barrier()
    for b in range(_NBUF):
        pltpu.sync_copy(x_hbm.at[pl.ds(base + b * _CHUNK, _CHUNK)],
                        idx_v.at[b])
        pltpu.sync_copy(pe_sh.at[pl.ds(phase(b), _CHUNK)], buf.at[b])
        pltpu.async_copy(w_hbm.at[idx_v.at[b]], buf.at[b], gsems[b],
                         add=True)

    def round_body(t, carry):
        for b in range(_NBUF):
            k = _NBUF * t + b
            rbase = base + k * _CHUNK
            pltpu.make_async_copy(w_hbm.at[idx_v.at[b]], buf.at[b],
                                  gsems[b]).wait()
            pltpu.async_copy(buf.at[b], out_hbm.at[pl.ds(rbase, _CHUNK)],
                             wsems[b])

            @pl.when(t < _NROUND - 1)
            def _prep_next():
                nbase = rbase + _NBUF * _CHUNK
                pltpu.async_copy(x_hbm.at[pl.ds(nbase, _CHUNK)],
                                 idx_v.at[b], isems[b])
                pltpu.make_async_copy(buf.at[b],
                                      out_hbm.at[pl.ds(rbase, _CHUNK)],
                                      wsems[b]).wait()
                pltpu.sync_copy(pe_sh.at[pl.ds(phase(k + _NBUF), _CHUNK)],
                                buf.at[b])
                pltpu.make_async_copy(x_hbm.at[pl.ds(nbase, _CHUNK)],
                                      idx_v.at[b], isems[b]).wait()
                pltpu.async_copy(w_hbm.at[idx_v.at[b]], buf.at[b],
                                 gsems[b], add=True)
        return carry

    lax.fori_loop(0, _NROUND, round_body, 0)
    for b in range(_NBUF):
        last = base + (_NCHUNK - _NBUF + b) * _CHUNK
        pltpu.make_async_copy(buf.at[b], out_hbm.at[pl.ds(last, _CHUNK)],
                              wsems[b]).wait()


def kernel(x, W):
    out = _embed_lookup(x.reshape(-1), W, _pe_table())
    return out.reshape(_B, _L, _EMBED)


# speedup vs baseline: 6.9509x; 1.0362x over previous
import functools

import jax
import jax.numpy as jnp
import numpy as np
from jax import lax
from jax.experimental import pallas as pl
from jax.experimental.pallas import tpu as pltpu
from jax.experimental.pallas import tpu_sc as plsc

_VOCAB = 100000
_EMBED = 128
_MAXLEN = 512
_B = 1024
_L = 200

_NC = 2
_NS = 16
_NW = _NC * _NS
_ROWS = _B * _L
_RPW = _ROWS // _NW
_CHUNK = 128
_NCHUNK = _RPW // _CHUNK
_NBUF = 5
_NROUND = _NCHUNK // _NBUF
_PE_PAD = _L + _CHUNK


def _pe_table() -> jnp.ndarray:
    pe = np.zeros((_MAXLEN, _EMBED), dtype=np.float32)
    position = np.arange(0, _MAXLEN)[:, np.newaxis]
    div_term = np.exp(np.arange(0, _EMBED, 2) * -(np.log(10000.0) / _EMBED))
    pe[:, 0::2] = np.sin(position * div_term)
    pe[:, 1::2] = np.cos(position * div_term)
    pe = pe[:_L]
    return jnp.asarray(np.concatenate([pe, pe[: _PE_PAD - _L]], axis=0))


_MESH = plsc.VectorSubcoreMesh(core_axis_name="c", subcore_axis_name="s")


@functools.partial(
    pl.kernel,
    out_type=jax.ShapeDtypeStruct((_ROWS, _EMBED), jnp.float32),
    mesh=_MESH,
    scratch_types=[
        pltpu.VMEM((_NBUF, _CHUNK), jnp.int32),
        pltpu.VMEM((_NBUF, _CHUNK, _EMBED), jnp.float32),
        pltpu.VMEM_SHARED((_PE_PAD, _EMBED), jnp.float32),
        [pltpu.SemaphoreType.DMA] * _NBUF,
        [pltpu.SemaphoreType.DMA] * _NBUF,
        [pltpu.SemaphoreType.DMA] * _NBUF,
    ],
)
def _embed_lookup(x_hbm, w_hbm, pe_hbm, out_hbm, idx_v, buf, pe_sh,
                  isems, gsems, wsems):
    wid = lax.axis_index("s") * _NC + lax.axis_index("c")
    base = wid * _RPW

    def phase(k):
        return lax.rem(k * _CHUNK, _L)

    @pl.when(lax.axis_index("s") == 0)
    def _stage_pe():
        pltpu.sync_copy(pe_hbm, pe_sh)

    plsc.subcore_barrier()
    for b in range(_NBUF):
        pltpu.sync_copy(x_hbm.at[pl.ds(base + b * _CHUNK, _CHUNK)],
                        idx_v.at[b])
        pltpu.sync_copy(pe_sh.at[pl.ds(phase(b), _CHUNK)], buf.at[b])
        pltpu.async_copy(w_hbm.at[idx_v.at[b]], buf.at[b], gsems[b],
                         add=True)

    def round_body(t, carry):
        for b in range(_NBUF):
            k = _NBUF * t + b
            rbase = base + k * _CHUNK
            pltpu.make_async_copy(w_hbm.at[idx_v.at[b]], buf.at[b],
                                  gsems[b]).wait()
            pltpu.async_copy(buf.at[b], out_hbm.at[pl.ds(rbase, _CHUNK)],
                             wsems[b])

            @pl.when(t < _NROUND - 1)
            def _prep_next():
                nbase = rbase + _NBUF * _CHUNK
                pltpu.async_copy(x_hbm.at[pl.ds(nbase, _CHUNK)],
                                 idx_v.at[b], isems[b])
                pltpu.make_async_copy(buf.at[b],
                                      out_hbm.at[pl.ds(rbase, _CHUNK)],
                                      wsems[b]).wait()
                pltpu.sync_copy(pe_sh.at[pl.ds(phase(k + _NBUF), _CHUNK)],
                                buf.at[b])
                pltpu.make_async_copy(x_hbm.at[pl.ds(nbase, _CHUNK)],
                                      idx_v.at[b], isems[b]).wait()
                pltpu.async_copy(w_hbm.at[idx_v.at[b]], buf.at[b],
                                 gsems[b], add=True)
        return carry

    lax.fori_loop(0, _NROUND, round_body, 0)
    for b in range(_NBUF):
        last = base + (_NCHUNK - _NBUF + b) * _CHUNK
        pltpu.make_async_copy(buf.at[b], out_hbm.at[pl.ds(last, _CHUNK)],
                              wsems[b]).wait()


def kernel(x, W):
    out = _embed_lookup(x.reshape(-1), W, _pe_table())
    return out.reshape(_B, _L, _EMBED)


# speedup vs baseline: 7.3413x; 1.0562x over previous
import functools

import jax
import jax.numpy as jnp
import numpy as np
from jax import lax
from jax.experimental import pallas as pl
from jax.experimental.pallas import tpu as pltpu
from jax.experimental.pallas import tpu_sc as plsc

_VOCAB = 100000
_EMBED = 128
_MAXLEN = 512
_B = 1024
_L = 200

_NC = 2
_NS = 16
_NW = _NC * _NS
_ROWS = _B * _L
_RPW = _ROWS // _NW
_CHUNK = 200
_NCHUNK = _RPW // _CHUNK
_NBUF = 4
_NROUND = _NCHUNK // _NBUF
_PE_PAD = _L + _CHUNK


def _pe_table() -> jnp.ndarray:
    pe = np.zeros((_MAXLEN, _EMBED), dtype=np.float32)
    position = np.arange(0, _MAXLEN)[:, np.newaxis]
    div_term = np.exp(np.arange(0, _EMBED, 2) * -(np.log(10000.0) / _EMBED))
    pe[:, 0::2] = np.sin(position * div_term)
    pe[:, 1::2] = np.cos(position * div_term)
    pe = pe[:_L]
    return jnp.asarray(np.concatenate([pe, pe[: _PE_PAD - _L]], axis=0))


_MESH = plsc.VectorSubcoreMesh(core_axis_name="c", subcore_axis_name="s")


@functools.partial(
    pl.kernel,
    out_type=jax.ShapeDtypeStruct((_ROWS, _EMBED), jnp.float32),
    mesh=_MESH,
    scratch_types=[
        pltpu.VMEM((_NBUF * _CHUNK,), jnp.int32),
        pltpu.VMEM((_NBUF, _CHUNK, _EMBED), jnp.float32),
        pltpu.VMEM_SHARED((_PE_PAD, _EMBED), jnp.float32),
        [pltpu.SemaphoreType.DMA] * _NBUF,
        [pltpu.SemaphoreType.DMA] * _NBUF,
        [pltpu.SemaphoreType.DMA] * _NBUF,
    ],
)
def _embed_lookup(x_hbm, w_hbm, pe_hbm, out_hbm, idx_v, buf, pe_sh,
                  isems, gsems, wsems):
    wid = lax.axis_index("s") * _NC + lax.axis_index("c")
    base = wid * _RPW

    def phase(k):
        return lax.rem(k * _CHUNK, _L)

    @pl.when(lax.axis_index("s") == 0)
    def _stage_pe():
        pltpu.sync_copy(pe_hbm, pe_sh)

    plsc.subcore_barrier()
    for b in range(_NBUF):
        pltpu.sync_copy(x_hbm.at[pl.ds(base + b * _CHUNK, _CHUNK)],
                        idx_v.at[pl.ds(b * _CHUNK, _CHUNK)])
        pltpu.sync_copy(pe_sh.at[pl.ds(phase(b), _CHUNK)], buf.at[b])
        pltpu.async_copy(w_hbm.at[idx_v.at[pl.ds(b * _CHUNK, _CHUNK)]], buf.at[b], gsems[b],
                         add=True)

    def round_body(t, carry):
        for b in range(_NBUF):
            k = _NBUF * t + b
            rbase = base + k * _CHUNK
            pltpu.make_async_copy(w_hbm.at[idx_v.at[pl.ds(b * _CHUNK, _CHUNK)]], buf.at[b],
                                  gsems[b]).wait()
            pltpu.async_copy(buf.at[b], out_hbm.at[pl.ds(rbase, _CHUNK)],
                             wsems[b])

            @pl.when(t < _NROUND - 1)
            def _prep_next():
                nbase = rbase + _NBUF * _CHUNK
                pltpu.async_copy(x_hbm.at[pl.ds(nbase, _CHUNK)],
                                 idx_v.at[pl.ds(b * _CHUNK, _CHUNK)], isems[b])
                pltpu.make_async_copy(buf.at[b],
                                      out_hbm.at[pl.ds(rbase, _CHUNK)],
                                      wsems[b]).wait()
                pltpu.sync_copy(pe_sh.at[pl.ds(phase(k + _NBUF), _CHUNK)],
                                buf.at[b])
                pltpu.make_async_copy(x_hbm.at[pl.ds(nbase, _CHUNK)],
                                      idx_v.at[pl.ds(b * _CHUNK, _CHUNK)], isems[b]).wait()
                pltpu.async_copy(w_hbm.at[idx_v.at[pl.ds(b * _CHUNK, _CHUNK)]], buf.at[b],
                                 gsems[b], add=True)
        return carry

    lax.fori_loop(0, _NROUND, round_body, 0)
    for b in range(_NBUF):
        last = base + (_NCHUNK - _NBUF + b) * _CHUNK
        pltpu.make_async_copy(buf.at[b], out_hbm.at[pl.ds(last, _CHUNK)],
                              wsems[b]).wait()


def kernel(x, W):
    out = _embed_lookup(x.reshape(-1), W, _pe_table())
    return out.reshape(_B, _L, _EMBED)
